# baseline (speedup 1.0000x reference)
import functools

import jax
import jax.numpy as jnp
from jax import lax
from jax.experimental import pallas as pl
from jax.experimental.pallas import tpu as pltpu
from jax.experimental.pallas import tpu_sc as plsc

MINI_BATCH = 16384
STATE_DIM = 64
NROW = 1000001
NC = 2
NS = 16
NW = NC * NS
B_PER_W = MINI_BATCH // NW
TW = 8192
TSTEPS = -(-NROW // TW)


def _pack_body(st_ref, nx_ref, out_ref):
    out_ref[:, 0:STATE_DIM] = st_ref[...].T
    out_ref[:, STATE_DIM:2 * STATE_DIM] = nx_ref[...].T


def _pack(state_t, next_t):
    return pl.pallas_call(
        _pack_body,
        grid=(TSTEPS,),
        out_shape=jax.ShapeDtypeStruct((NROW, 2 * STATE_DIM), jnp.float32),
        in_specs=[pl.BlockSpec((STATE_DIM, TW), lambda c: (0, c)),
                  pl.BlockSpec((STATE_DIM, TW), lambda c: (0, c))],
        out_specs=pl.BlockSpec((TW, 2 * STATE_DIM), lambda c: (c, 0)),
    )(state_t, next_t)


def _gather_body(comb_hbm, rew_hbm, msk_hbm, act_hbm, idx_hbm,
                 out_c, out_act, out_rew, out_msk,
                 idx_f, row_v, rew_v, msk_v, act_v, sem, sem3):
    wid = lax.axis_index("s") * NC + lax.axis_index("c")
    base = wid * B_PER_W

    pltpu.sync_copy(idx_hbm.at[pl.ds(base, B_PER_W)], idx_f)

    copies = [
        pltpu.async_copy(comb_hbm.at[idx_f], row_v, sem),
        pltpu.async_copy(rew_hbm.at[idx_f], rew_v, sem3),
        pltpu.async_copy(msk_hbm.at[idx_f], msk_v, sem3),
        pltpu.async_copy(act_hbm.at[idx_f], act_v, sem3),
    ]
    for cp in copies:
        cp.wait()

    pltpu.sync_copy(row_v, out_c.at[pl.ds(base, B_PER_W)])
    pltpu.sync_copy(rew_v, out_rew.at[pl.ds(base, B_PER_W)])
    pltpu.sync_copy(msk_v, out_msk.at[pl.ds(base, B_PER_W)])
    pltpu.sync_copy(act_v, out_act.at[pl.ds(base, B_PER_W)])


@jax.jit
def kernel(state, next_state, reward, masks, action, idx):
    idx = idx.astype(jnp.int32)
    act_dtype = action.dtype
    mesh = plsc.VectorSubcoreMesh(core_axis_name="c", subcore_axis_name="s")

    gather = pl.kernel(
        _gather_body,
        mesh=mesh,
        compiler_params=pltpu.CompilerParams(use_tc_tiling_on_sc=False),
        out_type=[
            jax.ShapeDtypeStruct((MINI_BATCH, 2 * STATE_DIM), jnp.float32),
            jax.ShapeDtypeStruct((MINI_BATCH,), act_dtype),
            jax.ShapeDtypeStruct((MINI_BATCH,), jnp.float32),
            jax.ShapeDtypeStruct((MINI_BATCH,), jnp.float32),
        ],
        scratch_types=[
            pltpu.VMEM((B_PER_W,), jnp.int32),
            pltpu.VMEM((B_PER_W, 2 * STATE_DIM), jnp.float32),
            pltpu.VMEM((B_PER_W,), jnp.float32),
            pltpu.VMEM((B_PER_W,), jnp.float32),
            pltpu.VMEM((B_PER_W,), act_dtype),
            pltpu.SemaphoreType.DMA,
            pltpu.SemaphoreType.DMA,
        ],
    )

    comb = _pack(state.T, next_state.T)
    out_c, out_act, out_rew, out_msk = gather(
        comb, reward, masks, action, idx)
    out_state = out_c[:, :STATE_DIM]
    out_next = out_c[:, STATE_DIM:]
    return (out_state, out_act, out_rew, out_next, out_msk)


# speedup vs baseline: 1.1213x; 1.1213x over previous
import functools

import jax
import jax.numpy as jnp
from jax import lax
from jax.experimental import pallas as pl
from jax.experimental.pallas import tpu as pltpu
from jax.experimental.pallas import tpu_sc as plsc

MINI_BATCH = 16384
STATE_DIM = 64
NROW = 1000001
NC = 2
NS = 16
NW = NC * NS
B_PER_W = MINI_BATCH // NW
NVEC = B_PER_W // 16

MAIN = 999936
PITCH = 1000064
TAIL_N = NROW - MAIN
TAIL_PAD = 8192
T0 = STATE_DIM * PITCH
FLATP = T0 + TAIL_PAD

FW = 142848
FC = MAIN // FW


def _flatten_body(st_ref, nx_ref, tail_st, tail_nx, dst_ref, sem, tsem):
    g = pl.program_id(0)
    c = pl.program_id(1)
    step = g * FC + c
    last = 8 * FC - 1

    tail_copies = [
        pltpu.make_async_copy(tail_st, dst_ref.at[pl.ds(T0, TAIL_PAD)],
                              tsem),
        pltpu.make_async_copy(tail_nx,
                              dst_ref.at[pl.ds(FLATP + T0, TAIL_PAD)], tsem),
    ]

    @pl.when(step == 0)
    def _():
        for tc in tail_copies:
            tc.start()

    for m in range(8):
        row = g * 8 + m
        pltpu.make_async_copy(
            st_ref.at[m],
            dst_ref.at[pl.ds(row * PITCH + c * FW, FW)], sem).start()
        pltpu.make_async_copy(
            nx_ref.at[m],
            dst_ref.at[pl.ds(FLATP + row * PITCH + c * FW, FW)], sem).start()

    for m in range(16):
        pltpu.make_async_copy(
            st_ref.at[m % 8], dst_ref.at[pl.ds(0, FW)], sem).wait()

    @pl.when(step == last)
    def _():
        for tc in tail_copies:
            tc.wait()


def _flatten_both(state_t, next_t, tail_st, tail_nx):
    return pl.pallas_call(
        _flatten_body,
        grid=(8, FC),
        out_shape=jax.ShapeDtypeStruct((2 * FLATP,), jnp.float32),
        in_specs=[pl.BlockSpec((8, FW), lambda g, c: (g, c)),
                  pl.BlockSpec((8, FW), lambda g, c: (g, c)),
                  pl.BlockSpec(memory_space=pltpu.MemorySpace.HBM),
                  pl.BlockSpec(memory_space=pltpu.MemorySpace.HBM)],
        out_specs=pl.BlockSpec(memory_space=pltpu.MemorySpace.HBM),
        scratch_shapes=[pltpu.SemaphoreType.DMA, pltpu.SemaphoreType.DMA],
    )(state_t, next_t, tail_st, tail_nx)


def _gather_all_body(flat_hbm, rew_hbm, msk_hbm, act_hbm, idx_hbm,
                     out_st, out_act, out_rew, out_nx, out_msk,
                     idx_f, off_f, st_c, nx_c, rew_v, msk_v, act_v,
                     sem, sem2, sem3):
    wid = lax.axis_index("s") * NC + lax.axis_index("c")
    base = wid * B_PER_W

    pltpu.sync_copy(idx_hbm.at[pl.ds(base, B_PER_W)], idx_f)

    copies = [
        pltpu.async_copy(rew_hbm.at[idx_f], rew_v, sem3),
        pltpu.async_copy(msk_hbm.at[idx_f], msk_v, sem3),
        pltpu.async_copy(act_hbm.at[idx_f], act_v, sem3),
    ]

    @pl.loop(0, STATE_DIM)
    def _off(j):
        c_main = j * PITCH
        c_tail = T0 + j * TAIL_N - MAIN
        for k in range(NVEC):
            v = idx_f[pl.ds(k * 16, 16)]
            off_f[pl.ds(j * B_PER_W + k * 16, 16)] = jnp.where(
                v < MAIN, v + c_main, v + c_tail)

    copies.append(pltpu.async_copy(
        flat_hbm.at[pl.ds(0, FLATP)].at[off_f], st_c, sem))
    copies.append(pltpu.async_copy(
        flat_hbm.at[pl.ds(FLATP, FLATP)].at[off_f], nx_c, sem2))
    for cp in copies:
        cp.wait()

    wcopies = []
    for j in range(STATE_DIM):
        wcopies.append(pltpu.async_copy(
            st_c.at[pl.ds(j * B_PER_W, B_PER_W)],
            out_st.at[j, pl.ds(base, B_PER_W)], sem))
        wcopies.append(pltpu.async_copy(
            nx_c.at[pl.ds(j * B_PER_W, B_PER_W)],
            out_nx.at[j, pl.ds(base, B_PER_W)], sem2))
    wcopies.append(pltpu.async_copy(rew_v, out_rew.at[pl.ds(base, B_PER_W)],
                                    sem3))
    wcopies.append(pltpu.async_copy(msk_v, out_msk.at[pl.ds(base, B_PER_W)],
                                    sem3))
    wcopies.append(pltpu.async_copy(act_v, out_act.at[pl.ds(base, B_PER_W)],
                                    sem3))
    for cp in wcopies:
        cp.wait()


def _tail(table):
    t = jnp.reshape(jnp.transpose(table[MAIN:, :]), (-1,))
    return jnp.pad(t, (0, TAIL_PAD - STATE_DIM * TAIL_N))


@jax.jit
def kernel(state, next_state, reward, masks, action, idx):
    idx = idx.astype(jnp.int32)
    act_dtype = action.dtype
    mesh = plsc.VectorSubcoreMesh(core_axis_name="c", subcore_axis_name="s")

    gather_all = pl.kernel(
        _gather_all_body,
        mesh=mesh,
        compiler_params=pltpu.CompilerParams(use_tc_tiling_on_sc=False),
        out_type=[
            jax.ShapeDtypeStruct((STATE_DIM, MINI_BATCH), jnp.float32),
            jax.ShapeDtypeStruct((MINI_BATCH,), act_dtype),
            jax.ShapeDtypeStruct((MINI_BATCH,), jnp.float32),
            jax.ShapeDtypeStruct((STATE_DIM, MINI_BATCH), jnp.float32),
            jax.ShapeDtypeStruct((MINI_BATCH,), jnp.float32),
        ],
        scratch_types=[
            pltpu.VMEM((B_PER_W,), jnp.int32),
            pltpu.VMEM((STATE_DIM * B_PER_W,), jnp.int32),
            pltpu.VMEM((STATE_DIM * B_PER_W,), jnp.float32),
            pltpu.VMEM((STATE_DIM * B_PER_W,), jnp.float32),
            pltpu.VMEM((B_PER_W,), jnp.float32),
            pltpu.VMEM((B_PER_W,), jnp.float32),
            pltpu.VMEM((B_PER_W,), act_dtype),
            pltpu.SemaphoreType.DMA,
            pltpu.SemaphoreType.DMA,
            pltpu.SemaphoreType.DMA,
        ],
    )

    flat = _flatten_both(state.T, next_state.T, _tail(state),
                         _tail(next_state))
    out_state_t, out_act, out_rew, out_next_t, out_msk = gather_all(
        flat, reward, masks, action, idx)
    return (out_state_t.T, out_act, out_rew, out_next_t.T, out_msk)


# speedup vs baseline: 1.1241x; 1.0025x over previous
import functools

import jax
import jax.numpy as jnp
from jax import lax
from jax.experimental import pallas as pl
from jax.experimental.pallas import tpu as pltpu
from jax.experimental.pallas import tpu_sc as plsc

MINI_BATCH = 16384
STATE_DIM = 64
NROW = 1000001
NC = 2
NS = 16
NW = NC * NS
B_PER_W = MINI_BATCH // NW
NVEC = B_PER_W // 16

MAIN = 999936
PITCH = 1000064
TAIL_N = NROW - MAIN
TAIL_PAD = 8192
T0 = STATE_DIM * PITCH
FLATP = T0 + TAIL_PAD

FW = 249984
FC = MAIN // FW


def _flatten_body(st_ref, nx_ref, tail_st, tail_nx, dst_ref, sem, tsem):
    g = pl.program_id(0)
    c = pl.program_id(1)
    step = g * FC + c
    last = 8 * FC - 1

    tail_copies = [
        pltpu.make_async_copy(tail_st, dst_ref.at[pl.ds(T0, TAIL_PAD)],
                              tsem),
        pltpu.make_async_copy(tail_nx,
                              dst_ref.at[pl.ds(FLATP + T0, TAIL_PAD)], tsem),
    ]

    @pl.when(step == 0)
    def _():
        for tc in tail_copies:
            tc.start()

    for m in range(8):
        row = g * 8 + m
        pltpu.make_async_copy(
            st_ref.at[m],
            dst_ref.at[pl.ds(row * PITCH + c * FW, FW)], sem).start()
        pltpu.make_async_copy(
            nx_ref.at[m],
            dst_ref.at[pl.ds(FLATP + row * PITCH + c * FW, FW)], sem).start()

    for m in range(16):
        pltpu.make_async_copy(
            st_ref.at[m % 8], dst_ref.at[pl.ds(0, FW)], sem).wait()

    @pl.when(step == last)
    def _():
        for tc in tail_copies:
            tc.wait()


def _flatten_both(state_t, next_t, tail_st, tail_nx):
    return pl.pallas_call(
        _flatten_body,
        grid=(8, FC),
        out_shape=jax.ShapeDtypeStruct((2 * FLATP,), jnp.float32),
        in_specs=[pl.BlockSpec((8, FW), lambda g, c: (g, c)),
                  pl.BlockSpec((8, FW), lambda g, c: (g, c)),
                  pl.BlockSpec(memory_space=pltpu.MemorySpace.HBM),
                  pl.BlockSpec(memory_space=pltpu.MemorySpace.HBM)],
        out_specs=pl.BlockSpec(memory_space=pltpu.MemorySpace.HBM),
        scratch_shapes=[pltpu.SemaphoreType.DMA, pltpu.SemaphoreType.DMA],
    )(state_t, next_t, tail_st, tail_nx)


def _gather_all_body(flat_hbm, rew_hbm, msk_hbm, act_hbm, idx_hbm,
                     out_st, out_act, out_rew, out_nx, out_msk,
                     idx_f, off_f, st_c, nx_c, rew_v, msk_v, act_v,
                     sem, sem2, sem3):
    wid = lax.axis_index("s") * NC + lax.axis_index("c")
    base = wid * B_PER_W

    pltpu.sync_copy(idx_hbm.at[pl.ds(base, B_PER_W)], idx_f)

    copies = [
        pltpu.async_copy(rew_hbm.at[idx_f], rew_v, sem3),
        pltpu.async_copy(msk_hbm.at[idx_f], msk_v, sem3),
        pltpu.async_copy(act_hbm.at[idx_f], act_v, sem3),
    ]

    @pl.loop(0, STATE_DIM)
    def _off(j):
        c_main = j * PITCH
        c_tail = T0 + j * TAIL_N - MAIN
        for k in range(NVEC):
            v = idx_f[pl.ds(k * 16, 16)]
            off_f[pl.ds(j * B_PER_W + k * 16, 16)] = jnp.where(
                v < MAIN, v + c_main, v + c_tail)

    copies.append(pltpu.async_copy(
        flat_hbm.at[pl.ds(0, FLATP)].at[off_f], st_c, sem))
    copies.append(pltpu.async_copy(
        flat_hbm.at[pl.ds(FLATP, FLATP)].at[off_f], nx_c, sem2))
    for cp in copies:
        cp.wait()

    wcopies = []
    for j in range(STATE_DIM):
        wcopies.append(pltpu.async_copy(
            st_c.at[pl.ds(j * B_PER_W, B_PER_W)],
            out_st.at[j, pl.ds(base, B_PER_W)], sem))
        wcopies.append(pltpu.async_copy(
            nx_c.at[pl.ds(j * B_PER_W, B_PER_W)],
            out_nx.at[j, pl.ds(base, B_PER_W)], sem2))
    wcopies.append(pltpu.async_copy(rew_v, out_rew.at[pl.ds(base, B_PER_W)],
                                    sem3))
    wcopies.append(pltpu.async_copy(msk_v, out_msk.at[pl.ds(base, B_PER_W)],
                                    sem3))
    wcopies.append(pltpu.async_copy(act_v, out_act.at[pl.ds(base, B_PER_W)],
                                    sem3))
    for cp in wcopies:
        cp.wait()


def _tail(table):
    t = jnp.reshape(jnp.transpose(table[MAIN:, :]), (-1,))
    return jnp.pad(t, (0, TAIL_PAD - STATE_DIM * TAIL_N))


@jax.jit
def kernel(state, next_state, reward, masks, action, idx):
    idx = idx.astype(jnp.int32)
    act_dtype = action.dtype
    mesh = plsc.VectorSubcoreMesh(core_axis_name="c", subcore_axis_name="s")

    gather_all = pl.kernel(
        _gather_all_body,
        mesh=mesh,
        compiler_params=pltpu.CompilerParams(use_tc_tiling_on_sc=False),
        out_type=[
            jax.ShapeDtypeStruct((STATE_DIM, MINI_BATCH), jnp.float32),
            jax.ShapeDtypeStruct((MINI_BATCH,), act_dtype),
            jax.ShapeDtypeStruct((MINI_BATCH,), jnp.float32),
            jax.ShapeDtypeStruct((STATE_DIM, MINI_BATCH), jnp.float32),
            jax.ShapeDtypeStruct((MINI_BATCH,), jnp.float32),
        ],
        scratch_types=[
            pltpu.VMEM((B_PER_W,), jnp.int32),
            pltpu.VMEM((STATE_DIM * B_PER_W,), jnp.int32),
            pltpu.VMEM((STATE_DIM * B_PER_W,), jnp.float32),
            pltpu.VMEM((STATE_DIM * B_PER_W,), jnp.float32),
            pltpu.VMEM((B_PER_W,), jnp.float32),
            pltpu.VMEM((B_PER_W,), jnp.float32),
            pltpu.VMEM((B_PER_W,), act_dtype),
            pltpu.SemaphoreType.DMA,
            pltpu.SemaphoreType.DMA,
            pltpu.SemaphoreType.DMA,
        ],
    )

    flat = _flatten_both(state.T, next_state.T, _tail(state),
                         _tail(next_state))
    out_state_t, out_act, out_rew, out_next_t, out_msk = gather_all(
        flat, reward, masks, action, idx)
    return (out_state_t.T, out_act, out_rew, out_next_t.T, out_msk)
